# baseline (speedup 1.0000x reference)
import functools

import jax
import jax.numpy as jnp
from jax import lax
from jax.experimental import pallas as pl
from jax.experimental.pallas import tpu as pltpu
from jax.experimental.pallas import tpu_sc as plsc

N = 10000
D = 128
NW = 32
CH = 128

@functools.cache
def _sc_mesh():
    return plsc.VectorSubcoreMesh(core_axis_name="c", subcore_axis_name="s")


def _affine(x, w, b, block_rows):
    rows, k = x.shape
    dout = w.shape[1]
    grid = (rows // block_rows,)

    def body(x_ref, w_ref, b_ref, o_ref):
        o_ref[...] = (
            jnp.dot(x_ref[...], w_ref[...], preferred_element_type=jnp.float32)
            + b_ref[...]
        )

    return pl.pallas_call(
        body,
        grid=grid,
        in_specs=[
            pl.BlockSpec((block_rows, k), lambda i: (i, 0)),
            pl.BlockSpec((k, dout), lambda i: (0, 0)),
            pl.BlockSpec((1, dout), lambda i: (0, 0)),
        ],
        out_specs=pl.BlockSpec((block_rows, dout), lambda i: (i, 0)),
        out_shape=jax.ShapeDtypeStruct((rows, dout), jnp.float32),
    )(x, w, b.reshape(1, dout))


def _proj2(x, wa, wb, bb):
    def body(x_ref, wa_ref, wb_ref, bb_ref, y1_ref, y2_ref):
        xv = x_ref[...]
        y1_ref[...] = jnp.dot(xv, wa_ref[...], preferred_element_type=jnp.float32)
        y2_ref[...] = (
            jnp.dot(xv, wb_ref[...], preferred_element_type=jnp.float32)
            + bb_ref[...]
        )

    return pl.pallas_call(
        body,
        out_shape=(
            jax.ShapeDtypeStruct((N, D), jnp.float32),
            jax.ShapeDtypeStruct((N, D), jnp.float32),
        ),
    )(x, wa, wb, bb.reshape(1, D))


def _node_update(x, agg2, w1, b1, w2, b2, g, b, wa, wb, bb):
    def body(x_ref, a_ref, w1_ref, b1_ref, w2_ref, b2_ref,
             g_ref, bb_ref, wa_ref, wb_ref, pb_ref, o_ref, y1_ref, y2_ref):
        xv = x_ref[...]
        h = xv + a_ref[0] + a_ref[1]
        h = jnp.maximum(
            jnp.dot(h, w1_ref[...], preferred_element_type=jnp.float32)
            + b1_ref[...],
            0.0,
        )
        h = jnp.dot(h, w2_ref[...], preferred_element_type=jnp.float32) + b2_ref[...]
        mu = jnp.mean(h, axis=0, keepdims=True)
        var = jnp.mean((h - mu) ** 2, axis=0, keepdims=True)
        h = (h - mu) / jnp.sqrt(var + 1e-5) * g_ref[...] + bb_ref[...]
        xn = (xv + jnp.maximum(h, 0.0)) * 0.5
        o_ref[...] = xn
        y1_ref[...] = jnp.dot(xn, wa_ref[...], preferred_element_type=jnp.float32)
        y2_ref[...] = (
            jnp.dot(xn, wb_ref[...], preferred_element_type=jnp.float32)
            + pb_ref[...]
        )

    return pl.pallas_call(
        body,
        out_shape=(
            jax.ShapeDtypeStruct((N, D), jnp.float32),
            jax.ShapeDtypeStruct((N, D), jnp.float32),
            jax.ShapeDtypeStruct((N, D), jnp.float32),
        ),
    )(x, agg2, w1, b1.reshape(1, D), w2, b2.reshape(1, D),
      g.reshape(1, D), b.reshape(1, D), wa, wb, bb.reshape(1, D))


def _edge_mlp(ea, z, gsum, w2, b2, block_rows):
    rows = ea.shape[0]
    grid = (rows // block_rows,)

    def body(ea_ref, z_ref, g_ref, w2_ref, b2_ref, o_ref):
        t = jnp.maximum(z_ref[...] + g_ref[...], 0.0)
        o_ref[...] = ea_ref[...] + (
            jnp.dot(t, w2_ref[...], preferred_element_type=jnp.float32)
            + b2_ref[...]
        ) * 0.5

    return pl.pallas_call(
        body,
        grid=grid,
        in_specs=[
            pl.BlockSpec((block_rows, D), lambda i: (i, 0)),
            pl.BlockSpec((block_rows, D), lambda i: (i, 0)),
            pl.BlockSpec((block_rows, D), lambda i: (i, 0)),
            pl.BlockSpec((D, D), lambda i: (0, 0)),
            pl.BlockSpec((1, D), lambda i: (0, 0)),
        ],
        out_specs=pl.BlockSpec((block_rows, D), lambda i: (i, 0)),
        out_shape=jax.ShapeDtypeStruct((rows, D), jnp.float32),
    )(ea, z, gsum, w2, b2.reshape(1, D))


def _head_mlp(zh, gh, w2, b2, block_rows):
    rows = zh.shape[0]
    nc = w2.shape[1]
    grid = (rows // block_rows,)

    def body(z_ref, g_ref, w2_ref, b2_ref, o_ref):
        t = jnp.maximum(z_ref[...] + g_ref[...], 0.0)
        o_ref[...] = (
            jnp.dot(t, w2_ref[...], preferred_element_type=jnp.float32)
            + b2_ref[...]
        )

    return pl.pallas_call(
        body,
        grid=grid,
        in_specs=[
            pl.BlockSpec((block_rows, D), lambda i: (i, 0)),
            pl.BlockSpec((block_rows, D), lambda i: (i, 0)),
            pl.BlockSpec((D, nc), lambda i: (0, 0)),
            pl.BlockSpec((1, nc), lambda i: (0, 0)),
        ],
        out_specs=pl.BlockSpec((block_rows, nc), lambda i: (i, 0)),
        out_shape=jax.ShapeDtypeStruct((rows, nc), jnp.float32),
    )(zh, gh, w2, b2.reshape(1, nc))


def _agg_xla(x1, ea, src_mat, dst_mat, zeros):
    src = src_mat.reshape(-1)
    dst = dst_mat.reshape(-1)
    msg = jnp.maximum(x1[src] + ea, 0.0)
    a = jax.ops.segment_sum(msg, dst, num_segments=N)
    return jnp.stack([a, jnp.zeros_like(a)])


def _agg(x1, ea, src_mat, dst_mat, zeros):
    nchunks = src_mat.shape[0]
    CA = src_mat.shape[1]
    niter = (nchunks + NW - 1) // NW
    RZ = 624
    TAIL = N - 16 * RZ

    @functools.partial(
        pl.kernel,
        out_type=jax.ShapeDtypeStruct((2, N, D), jnp.float32),
        mesh=_sc_mesh(),
        scratch_types=[
            pltpu.VMEM((2, 1, CA), jnp.int32),
            pltpu.VMEM((2, 1, CA), jnp.int32),
            pltpu.VMEM((2, CA, D), jnp.float32),
            pltpu.VMEM((2, CA, D), jnp.float32),
            pltpu.VMEM_SHARED((N, D), jnp.float32),
            pltpu.SemaphoreType.DMA,
            pltpu.SemaphoreType.DMA,
        ],
    )
    def k(x_hbm, ea_hbm, src_hbm, dst_hbm, z_hbm, out_hbm,
          idxs, idxd, bufx, bufe, agg_sh, sga, sgb):
        c = lax.axis_index("c")
        s = lax.axis_index("s")
        wid = s * 2 + c

        pltpu.sync_copy(z_hbm.at[pl.ds(s * RZ, RZ)], agg_sh.at[pl.ds(s * RZ, RZ)])

        @pl.when(s == 0)
        def _():
            pltpu.sync_copy(z_hbm.at[pl.ds(16 * RZ, TAIL)],
                            agg_sh.at[pl.ds(16 * RZ, TAIL)])

        plsc.subcore_barrier()

        def chunk_of(j):
            return wid + j * NW

        def start_in(j, slot, sem):
            ck = chunk_of(j)
            pltpu.sync_copy(src_hbm.at[pl.ds(ck, 1)], idxs.at[slot])
            pltpu.sync_copy(dst_hbm.at[pl.ds(ck, 1)], idxd.at[slot])
            pltpu.async_copy(x_hbm.at[idxs.at[slot, 0]], bufx.at[slot], sem)
            pltpu.async_copy(ea_hbm.at[pl.ds(ck * CA, CA)], bufe.at[slot], sem)

        def finish(j, slot, sem):
            ck = chunk_of(j)
            pltpu.make_async_copy(x_hbm.at[idxs.at[slot, 0]],
                                  bufx.at[slot], sem).wait()
            pltpu.make_async_copy(ea_hbm.at[pl.ds(ck * CA, CA)],
                                  bufe.at[slot], sem).wait()
            _add_rows(bufx.at[slot], bufx.at[slot], bufe.at[slot],
                      relu=True, ch=CA)
            pltpu.sync_copy(bufx.at[slot], agg_sh.at[idxd.at[slot, 0]], add=True)

        @pl.when(chunk_of(0) < nchunks)
        def _():
            start_in(0, 0, sga)

        @pl.loop(0, (niter + 1) // 2)
        def _(p):
            j0 = 2 * p
            j1 = j0 + 1

            @pl.when(chunk_of(j1) < nchunks)
            def _():
                start_in(j1, 1, sgb)

            @pl.when(chunk_of(j0) < nchunks)
            def _():
                finish(j0, 0, sga)

            @pl.when(chunk_of(j0 + 2) < nchunks)
            def _():
                start_in(j0 + 2, 0, sga)

            @pl.when(chunk_of(j1) < nchunks)
            def _():
                finish(j1, 1, sgb)

        plsc.subcore_barrier()

        pltpu.sync_copy(agg_sh.at[pl.ds(s * RZ, RZ)],
                        out_hbm.at[c, pl.ds(s * RZ, RZ)])

        @pl.when(s == 0)
        def _():
            pltpu.sync_copy(agg_sh.at[pl.ds(16 * RZ, TAIL)],
                            out_hbm.at[c, pl.ds(16 * RZ, TAIL)])

    return k(x1, ea, src_mat, dst_mat, zeros)


def _pad_chunks(mat, maxc):
    nchunks = mat.shape[0]
    pad = NW * maxc - nchunks
    if pad:
        mat = jnp.pad(mat, ((0, pad), (0, 0)))
    return mat


def _add_rows(dst, a, b, relu, ch):
    @pl.loop(0, ch)
    def _(r):
        for cc in range(D // 16):
            sl = (pl.ds(r, 1), pl.ds(cc * 16, 16))
            v = a.at[sl][...] + b.at[sl][...]
            if relu:
                v = jnp.maximum(v, 0.0)
            dst.at[sl][...] = v


def _gather_add2_xla(y1, y2, a_mat, b_mat):
    a = a_mat.reshape(-1)
    b = b_mat.reshape(-1)
    return y1[a] + y2[b]


def _gather_add2(y1, y2, a_mat, b_mat):
    nchunks = a_mat.shape[0]
    rows = nchunks * CH
    maxc = (-(-nchunks // NW) + 7) // 8 * 8
    a_mat = _pad_chunks(a_mat, maxc)
    b_mat = _pad_chunks(b_mat, maxc)

    @functools.partial(
        pl.kernel,
        out_type=jax.ShapeDtypeStruct((rows, D), jnp.float32),
        mesh=_sc_mesh(),
        scratch_types=[
            pltpu.VMEM((maxc, CH), jnp.int32),
            pltpu.VMEM((maxc, CH), jnp.int32),
            pltpu.VMEM((2, CH, D), jnp.float32),
            pltpu.VMEM((2, CH, D), jnp.float32),
            pltpu.VMEM((2, CH, D), jnp.float32),
            pltpu.SemaphoreType.DMA,
            pltpu.SemaphoreType.DMA,
            pltpu.SemaphoreType.DMA,
            pltpu.SemaphoreType.DMA,
        ],
    )
    def k(y1_hbm, y2_hbm, a_hbm, b_hbm, out_hbm,
          idxa, idxb, bufa, bufb, bufo, sga, sgb, soa, sob):
        wid = lax.axis_index("s") * 2 + lax.axis_index("c")
        base = wid * maxc
        cnt = jnp.maximum(jnp.minimum(nchunks - base, maxc), 0)
        pltpu.sync_copy(a_hbm.at[pl.ds(base, maxc)], idxa)
        pltpu.sync_copy(b_hbm.at[pl.ds(base, maxc)], idxb)

        def start_gathers(j, slot, sem):
            pltpu.async_copy(y1_hbm.at[idxa.at[j]], bufa.at[slot], sem)
            pltpu.async_copy(y2_hbm.at[idxb.at[j]], bufb.at[slot], sem)

        def wait_gathers(j, slot, sem):
            pltpu.make_async_copy(y1_hbm.at[idxa.at[j]], bufa.at[slot], sem).wait()
            pltpu.make_async_copy(y2_hbm.at[idxb.at[j]], bufb.at[slot], sem).wait()

        def out_copy(j, slot, sem):
            return pltpu.make_async_copy(
                bufo.at[slot], out_hbm.at[pl.ds((base + j) * CH, CH)], sem)

        @pl.when(cnt > 0)
        def _():
            start_gathers(0, 0, sga)

        @pl.loop(0, maxc // 2)
        def _(p):
            j0 = 2 * p
            j1 = j0 + 1

            @pl.when(j1 < cnt)
            def _():
                start_gathers(j1, 1, sgb)

            @pl.when(j0 < cnt)
            def _():
                wait_gathers(j0, 0, sga)

                @pl.when(p > 0)
                def _():
                    out_copy(j0 - 2, 0, soa).wait()

                _add_rows(bufo.at[0], bufa.at[0], bufb.at[0], relu=False, ch=CH)
                out_copy(j0, 0, soa).start()

            @pl.when(j0 + 2 < cnt)
            def _():
                start_gathers(j0 + 2, 0, sga)

            @pl.when(j1 < cnt)
            def _():
                wait_gathers(j1, 1, sgb)

                @pl.when(p > 0)
                def _():
                    out_copy(j1 - 2, 1, sob).wait()

                _add_rows(bufo.at[1], bufa.at[1], bufb.at[1], relu=False, ch=CH)
                out_copy(j1, 1, sob).start()

        @pl.when(cnt > 0)
        def _():
            out_copy(cnt - 1 - (cnt + 1) % 2, 0, soa).wait()

        @pl.when(cnt > 1)
        def _():
            out_copy(cnt - 1 - cnt % 2, 1, sob).wait()

    return k(y1, y2, a_mat, b_mat)


def kernel(x, edge_index, edge_attr, pos_edge_index, pos_edge_attr,
           neg_edge_index, neg_edge_attr,
           node_W, node_b, edge_W, edge_b,
           convW1, convb1, convW2, convb2, bnG, bnB,
           emlpW1, emlpb1, emlpW2, emlpb2,
           decW1, decb1, decW2, decb2):
    E = edge_attr.shape[0]
    L = convW1.shape[0]

    CA = 80
    src_mat = edge_index[0].reshape(E // CH, CH).astype(jnp.int32)
    dst_mat = edge_index[1].reshape(E // CH, CH).astype(jnp.int32)
    srca_mat = edge_index[0].reshape(E // CA, CA).astype(jnp.int32)
    dsta_mat = edge_index[1].reshape(E // CA, CA).astype(jnp.int32)
    pn_index = jnp.concatenate([pos_edge_index, neg_edge_index], axis=1)
    EH = pn_index.shape[1]
    pn_a = pn_index[0].reshape(EH // CH, CH).astype(jnp.int32)
    pn_b = pn_index[1].reshape(EH // CH, CH).astype(jnp.int32)
    zeros = jnp.zeros((N, D), jnp.float32)

    x1 = _affine(x, node_W, node_b, 2000)
    ea = _affine(edge_attr, edge_W, edge_b, 3200)
    pn_attr = jnp.concatenate([pos_edge_attr, neg_edge_attr], axis=0)
    pn_ea = _affine(pn_attr, edge_W, edge_b, 4096)

    zero_b = jnp.zeros((D,), jnp.float32)
    for i in range(L):
        last = i + 1 == L
        wa, wb, bb = ((decW1[0:D], decW1[D:2 * D], decb1) if last else
                      (emlpW1[i][0:D], emlpW1[i][D:2 * D], emlpb1[i]))
        agg2 = _agg(x1, ea, srca_mat, dsta_mat, zeros)
        x1, y1, y2 = _node_update(x1, agg2, convW1[i], convb1[i],
                                  convW2[i], convb2[i], bnG[i], bnB[i],
                                  wa, wb, bb)
        if last:
            break
        z = _affine(ea, emlpW1[i][2 * D:3 * D], zero_b, 3200)
        gsum = _gather_add2(y1, y2, src_mat, dst_mat)
        ea = _edge_mlp(ea, z, gsum, emlpW2[i], emlpb2[i], 3200)

    zh = _affine(pn_ea, decW1[2 * D:3 * D], zero_b, 4096)
    gh = _gather_add2(y1, y2, pn_a, pn_b)
    return _head_mlp(zh, gh, decW2, decb2, 4096)


# speedup vs baseline: 1.1190x; 1.1190x over previous
import functools

import jax
import jax.numpy as jnp
from jax import lax
from jax.experimental import pallas as pl
from jax.experimental.pallas import tpu as pltpu
from jax.experimental.pallas import tpu_sc as plsc

N = 10000
D = 128
NW = 32
CH = 128

@functools.cache
def _sc_mesh():
    return plsc.VectorSubcoreMesh(core_axis_name="c", subcore_axis_name="s")


def _affine(x, w, b, block_rows):
    rows, k = x.shape
    dout = w.shape[1]
    grid = (rows // block_rows,)

    def body(x_ref, w_ref, b_ref, o_ref):
        o_ref[...] = (
            jnp.dot(x_ref[...], w_ref[...], preferred_element_type=jnp.float32)
            + b_ref[...]
        )

    return pl.pallas_call(
        body,
        grid=grid,
        in_specs=[
            pl.BlockSpec((block_rows, k), lambda i: (i, 0)),
            pl.BlockSpec((k, dout), lambda i: (0, 0)),
            pl.BlockSpec((1, dout), lambda i: (0, 0)),
        ],
        out_specs=pl.BlockSpec((block_rows, dout), lambda i: (i, 0)),
        out_shape=jax.ShapeDtypeStruct((rows, dout), jnp.float32),
    )(x, w, b.reshape(1, dout))


def _proj2(x, wa, wb, bb):
    def body(x_ref, wa_ref, wb_ref, bb_ref, y1_ref, y2_ref):
        xv = x_ref[...]
        y1_ref[...] = jnp.dot(xv, wa_ref[...], preferred_element_type=jnp.float32)
        y2_ref[...] = (
            jnp.dot(xv, wb_ref[...], preferred_element_type=jnp.float32)
            + bb_ref[...]
        )

    return pl.pallas_call(
        body,
        out_shape=(
            jax.ShapeDtypeStruct((N, D), jnp.float32),
            jax.ShapeDtypeStruct((N, D), jnp.float32),
        ),
    )(x, wa, wb, bb.reshape(1, D))


def _node_update(x, agg2, w1, b1, w2, b2, g, b, wa, wb, bb):
    def body(x_ref, a_ref, w1_ref, b1_ref, w2_ref, b2_ref,
             g_ref, bb_ref, wa_ref, wb_ref, pb_ref, o_ref, y1_ref, y2_ref):
        xv = x_ref[...]
        h = xv + a_ref[0] + a_ref[1]
        h = jnp.maximum(
            jnp.dot(h, w1_ref[...], preferred_element_type=jnp.float32)
            + b1_ref[...],
            0.0,
        )
        h = jnp.dot(h, w2_ref[...], preferred_element_type=jnp.float32) + b2_ref[...]
        mu = jnp.mean(h, axis=0, keepdims=True)
        var = jnp.mean((h - mu) ** 2, axis=0, keepdims=True)
        h = (h - mu) / jnp.sqrt(var + 1e-5) * g_ref[...] + bb_ref[...]
        xn = (xv + jnp.maximum(h, 0.0)) * 0.5
        o_ref[...] = xn
        y1_ref[...] = jnp.dot(xn, wa_ref[...], preferred_element_type=jnp.float32)
        y2_ref[...] = (
            jnp.dot(xn, wb_ref[...], preferred_element_type=jnp.float32)
            + pb_ref[...]
        )

    return pl.pallas_call(
        body,
        out_shape=(
            jax.ShapeDtypeStruct((N, D), jnp.float32),
            jax.ShapeDtypeStruct((N, D), jnp.float32),
            jax.ShapeDtypeStruct((N, D), jnp.float32),
        ),
    )(x, agg2, w1, b1.reshape(1, D), w2, b2.reshape(1, D),
      g.reshape(1, D), b.reshape(1, D), wa, wb, bb.reshape(1, D))


def _edge_mlp(ea, gsum, w1e, w2, b2, block_rows):
    rows = ea.shape[0]
    grid = (rows // block_rows,)

    def body(ea_ref, g_ref, w1e_ref, w2_ref, b2_ref, o_ref):
        eav = ea_ref[...]
        t = jnp.maximum(
            g_ref[...]
            + jnp.dot(eav, w1e_ref[...], preferred_element_type=jnp.float32),
            0.0,
        )
        o_ref[...] = eav + (
            jnp.dot(t, w2_ref[...], preferred_element_type=jnp.float32)
            + b2_ref[...]
        ) * 0.5

    return pl.pallas_call(
        body,
        grid=grid,
        in_specs=[
            pl.BlockSpec((block_rows, D), lambda i: (i, 0)),
            pl.BlockSpec((block_rows, D), lambda i: (i, 0)),
            pl.BlockSpec((D, D), lambda i: (0, 0)),
            pl.BlockSpec((D, D), lambda i: (0, 0)),
            pl.BlockSpec((1, D), lambda i: (0, 0)),
        ],
        out_specs=pl.BlockSpec((block_rows, D), lambda i: (i, 0)),
        out_shape=jax.ShapeDtypeStruct((rows, D), jnp.float32),
    )(ea, gsum, w1e, w2, b2.reshape(1, D))


def _head_mlp(eah, gh, w1e, w2, b2, block_rows):
    rows = eah.shape[0]
    nc = w2.shape[1]
    grid = (rows // block_rows,)

    def body(ea_ref, g_ref, w1e_ref, w2_ref, b2_ref, o_ref):
        t = jnp.maximum(
            g_ref[...]
            + jnp.dot(ea_ref[...], w1e_ref[...], preferred_element_type=jnp.float32),
            0.0,
        )
        o_ref[...] = (
            jnp.dot(t, w2_ref[...], preferred_element_type=jnp.float32)
            + b2_ref[...]
        )

    return pl.pallas_call(
        body,
        grid=grid,
        in_specs=[
            pl.BlockSpec((block_rows, D), lambda i: (i, 0)),
            pl.BlockSpec((block_rows, D), lambda i: (i, 0)),
            pl.BlockSpec((D, D), lambda i: (0, 0)),
            pl.BlockSpec((D, nc), lambda i: (0, 0)),
            pl.BlockSpec((1, nc), lambda i: (0, 0)),
        ],
        out_specs=pl.BlockSpec((block_rows, nc), lambda i: (i, 0)),
        out_shape=jax.ShapeDtypeStruct((rows, nc), jnp.float32),
    )(eah, gh, w1e, w2, b2.reshape(1, nc))


def _agg_xla(x1, ea, src_mat, dst_mat, zeros):
    src = src_mat.reshape(-1)
    dst = dst_mat.reshape(-1)
    msg = jnp.maximum(x1[src] + ea, 0.0)
    a = jax.ops.segment_sum(msg, dst, num_segments=N)
    return jnp.stack([a, jnp.zeros_like(a)])


def _agg(x1, ea, src_mat, dst_mat, zeros):
    nchunks = src_mat.shape[0]
    CA = src_mat.shape[1]
    niter = (nchunks + NW - 1) // NW
    RZ = 624
    TAIL = N - 16 * RZ

    @functools.partial(
        pl.kernel,
        out_type=jax.ShapeDtypeStruct((2, N, D), jnp.float32),
        mesh=_sc_mesh(),
        scratch_types=[
            pltpu.VMEM((2, 1, CA), jnp.int32),
            pltpu.VMEM((2, 1, CA), jnp.int32),
            pltpu.VMEM((2, CA, D), jnp.float32),
            pltpu.VMEM((2, CA, D), jnp.float32),
            pltpu.VMEM_SHARED((N, D), jnp.float32),
            pltpu.SemaphoreType.DMA,
            pltpu.SemaphoreType.DMA,
        ],
    )
    def k(x_hbm, ea_hbm, src_hbm, dst_hbm, z_hbm, out_hbm,
          idxs, idxd, bufx, bufe, agg_sh, sga, sgb):
        c = lax.axis_index("c")
        s = lax.axis_index("s")
        wid = s * 2 + c

        pltpu.sync_copy(z_hbm.at[pl.ds(s * RZ, RZ)], agg_sh.at[pl.ds(s * RZ, RZ)])

        @pl.when(s == 0)
        def _():
            pltpu.sync_copy(z_hbm.at[pl.ds(16 * RZ, TAIL)],
                            agg_sh.at[pl.ds(16 * RZ, TAIL)])

        plsc.subcore_barrier()

        def chunk_of(j):
            return wid + j * NW

        def start_in(j, slot, sem):
            ck = chunk_of(j)
            pltpu.sync_copy(src_hbm.at[pl.ds(ck, 1)], idxs.at[slot])
            pltpu.sync_copy(dst_hbm.at[pl.ds(ck, 1)], idxd.at[slot])
            pltpu.async_copy(x_hbm.at[idxs.at[slot, 0]], bufx.at[slot], sem)
            pltpu.async_copy(ea_hbm.at[pl.ds(ck * CA, CA)], bufe.at[slot], sem)

        def finish(j, slot, sem):
            ck = chunk_of(j)
            pltpu.make_async_copy(x_hbm.at[idxs.at[slot, 0]],
                                  bufx.at[slot], sem).wait()
            pltpu.make_async_copy(ea_hbm.at[pl.ds(ck * CA, CA)],
                                  bufe.at[slot], sem).wait()
            _add_rows(bufx.at[slot], bufx.at[slot], bufe.at[slot],
                      relu=True, ch=CA)
            pltpu.sync_copy(bufx.at[slot], agg_sh.at[idxd.at[slot, 0]], add=True)

        @pl.when(chunk_of(0) < nchunks)
        def _():
            start_in(0, 0, sga)

        @pl.loop(0, (niter + 1) // 2)
        def _(p):
            j0 = 2 * p
            j1 = j0 + 1

            @pl.when(chunk_of(j1) < nchunks)
            def _():
                start_in(j1, 1, sgb)

            @pl.when(chunk_of(j0) < nchunks)
            def _():
                finish(j0, 0, sga)

            @pl.when(chunk_of(j0 + 2) < nchunks)
            def _():
                start_in(j0 + 2, 0, sga)

            @pl.when(chunk_of(j1) < nchunks)
            def _():
                finish(j1, 1, sgb)

        plsc.subcore_barrier()

        pltpu.sync_copy(agg_sh.at[pl.ds(s * RZ, RZ)],
                        out_hbm.at[c, pl.ds(s * RZ, RZ)])

        @pl.when(s == 0)
        def _():
            pltpu.sync_copy(agg_sh.at[pl.ds(16 * RZ, TAIL)],
                            out_hbm.at[c, pl.ds(16 * RZ, TAIL)])

    return k(x1, ea, src_mat, dst_mat, zeros)


def _pad_chunks(mat, maxc):
    nchunks = mat.shape[0]
    pad = NW * maxc - nchunks
    if pad:
        mat = jnp.pad(mat, ((0, pad), (0, 0)))
    return mat


def _add_rows(dst, a, b, relu, ch):
    @pl.loop(0, ch)
    def _(r):
        for cc in range(D // 16):
            sl = (pl.ds(r, 1), pl.ds(cc * 16, 16))
            v = a.at[sl][...] + b.at[sl][...]
            if relu:
                v = jnp.maximum(v, 0.0)
            dst.at[sl][...] = v


def _gather_add2_xla(y1, y2, a_mat, b_mat):
    a = a_mat.reshape(-1)
    b = b_mat.reshape(-1)
    return y1[a] + y2[b]


def _gather_add2(y1, y2, a_mat, b_mat):
    nchunks = a_mat.shape[0]
    rows = nchunks * CH
    maxc = (-(-nchunks // NW) + 7) // 8 * 8
    a_mat = _pad_chunks(a_mat, maxc)
    b_mat = _pad_chunks(b_mat, maxc)

    @functools.partial(
        pl.kernel,
        out_type=jax.ShapeDtypeStruct((rows, D), jnp.float32),
        mesh=_sc_mesh(),
        scratch_types=[
            pltpu.VMEM((maxc, CH), jnp.int32),
            pltpu.VMEM((maxc, CH), jnp.int32),
            pltpu.VMEM((2, CH, D), jnp.float32),
            pltpu.VMEM((2, CH, D), jnp.float32),
            pltpu.VMEM((2, CH, D), jnp.float32),
            pltpu.SemaphoreType.DMA,
            pltpu.SemaphoreType.DMA,
            pltpu.SemaphoreType.DMA,
            pltpu.SemaphoreType.DMA,
        ],
    )
    def k(y1_hbm, y2_hbm, a_hbm, b_hbm, out_hbm,
          idxa, idxb, bufa, bufb, bufo, sga, sgb, soa, sob):
        wid = lax.axis_index("s") * 2 + lax.axis_index("c")
        base = wid * maxc
        cnt = jnp.maximum(jnp.minimum(nchunks - base, maxc), 0)
        pltpu.sync_copy(a_hbm.at[pl.ds(base, maxc)], idxa)
        pltpu.sync_copy(b_hbm.at[pl.ds(base, maxc)], idxb)

        def start_gathers(j, slot, sem):
            pltpu.async_copy(y1_hbm.at[idxa.at[j]], bufa.at[slot], sem)
            pltpu.async_copy(y2_hbm.at[idxb.at[j]], bufb.at[slot], sem)

        def wait_gathers(j, slot, sem):
            pltpu.make_async_copy(y1_hbm.at[idxa.at[j]], bufa.at[slot], sem).wait()
            pltpu.make_async_copy(y2_hbm.at[idxb.at[j]], bufb.at[slot], sem).wait()

        def out_copy(j, slot, sem):
            return pltpu.make_async_copy(
                bufo.at[slot], out_hbm.at[pl.ds((base + j) * CH, CH)], sem)

        @pl.when(cnt > 0)
        def _():
            start_gathers(0, 0, sga)

        @pl.loop(0, maxc // 2)
        def _(p):
            j0 = 2 * p
            j1 = j0 + 1

            @pl.when(j1 < cnt)
            def _():
                start_gathers(j1, 1, sgb)

            @pl.when(j0 < cnt)
            def _():
                wait_gathers(j0, 0, sga)

                @pl.when(p > 0)
                def _():
                    out_copy(j0 - 2, 0, soa).wait()

                _add_rows(bufo.at[0], bufa.at[0], bufb.at[0], relu=False, ch=CH)
                out_copy(j0, 0, soa).start()

            @pl.when(j0 + 2 < cnt)
            def _():
                start_gathers(j0 + 2, 0, sga)

            @pl.when(j1 < cnt)
            def _():
                wait_gathers(j1, 1, sgb)

                @pl.when(p > 0)
                def _():
                    out_copy(j1 - 2, 1, sob).wait()

                _add_rows(bufo.at[1], bufa.at[1], bufb.at[1], relu=False, ch=CH)
                out_copy(j1, 1, sob).start()

        @pl.when(cnt > 0)
        def _():
            out_copy(cnt - 1 - (cnt + 1) % 2, 0, soa).wait()

        @pl.when(cnt > 1)
        def _():
            out_copy(cnt - 1 - cnt % 2, 1, sob).wait()

    return k(y1, y2, a_mat, b_mat)


def kernel(x, edge_index, edge_attr, pos_edge_index, pos_edge_attr,
           neg_edge_index, neg_edge_attr,
           node_W, node_b, edge_W, edge_b,
           convW1, convb1, convW2, convb2, bnG, bnB,
           emlpW1, emlpb1, emlpW2, emlpb2,
           decW1, decb1, decW2, decb2):
    E = edge_attr.shape[0]
    L = convW1.shape[0]

    CA = 80
    src_mat = edge_index[0].reshape(E // CH, CH).astype(jnp.int32)
    dst_mat = edge_index[1].reshape(E // CH, CH).astype(jnp.int32)
    srca_mat = edge_index[0].reshape(E // CA, CA).astype(jnp.int32)
    dsta_mat = edge_index[1].reshape(E // CA, CA).astype(jnp.int32)
    pn_index = jnp.concatenate([pos_edge_index, neg_edge_index], axis=1)
    EH = pn_index.shape[1]
    pn_a = pn_index[0].reshape(EH // CH, CH).astype(jnp.int32)
    pn_b = pn_index[1].reshape(EH // CH, CH).astype(jnp.int32)
    zeros = jnp.zeros((N, D), jnp.float32)

    x1 = _affine(x, node_W, node_b, 2000)
    ea = _affine(edge_attr, edge_W, edge_b, 3200)
    pn_attr = jnp.concatenate([pos_edge_attr, neg_edge_attr], axis=0)
    pn_ea = _affine(pn_attr, edge_W, edge_b, 4096)

    for i in range(L):
        last = i + 1 == L
        wa, wb, bb = ((decW1[0:D], decW1[D:2 * D], decb1) if last else
                      (emlpW1[i][0:D], emlpW1[i][D:2 * D], emlpb1[i]))
        agg2 = _agg(x1, ea, srca_mat, dsta_mat, zeros)
        x1, y1, y2 = _node_update(x1, agg2, convW1[i], convb1[i],
                                  convW2[i], convb2[i], bnG[i], bnB[i],
                                  wa, wb, bb)
        if last:
            break
        gsum = _gather_add2(y1, y2, src_mat, dst_mat)
        ea = _edge_mlp(ea, gsum, emlpW1[i][2 * D:3 * D], emlpW2[i],
                       emlpb2[i], 3200)

    gh = _gather_add2(y1, y2, pn_a, pn_b)
    return _head_mlp(pn_ea, gh, decW1[2 * D:3 * D], decW2, decb2, 4096)
